# baseline (speedup 1.0000x reference)
import functools

import jax
import jax.numpy as jnp
from jax import lax
from jax.experimental import pallas as pl
from jax.experimental.pallas import tpu as pltpu
from jax.experimental.pallas import tpu_sc as plsc

_LANES = 16

_GATHER_DNUMS = lax.GatherDimensionNumbers(
    offset_dims=(), collapsed_slice_dims=(0,), start_index_map=(0,))


def _take16(vec, idx):
    return lax.gather(vec, idx[:, None], _GATHER_DNUMS, (1,),
                      mode=lax.GatherScatterMode.PROMISE_IN_BOUNDS)


@functools.lru_cache(maxsize=None)
def _build_gather_kernel(B, N, H, W):
    info = plsc.get_sparse_core_info()
    NC, NS = info.num_cores, info.num_subcores
    NW = NC * NS
    HW = H * W
    TOTAL = B * N
    assert TOTAL % NW == 0
    per_tile = TOTAL // NW

    C = min(12800, per_tile)
    assert C % _LANES == 0
    bases = [j * C for j in range(per_tile // C)]
    if per_tile % C:
        bases.append(per_tile - C)
    assert all(b % 8 == 0 for b in bases)

    mesh = plsc.VectorSubcoreMesh(core_axis_name="c", subcore_axis_name="s")

    @functools.partial(
        pl.kernel,
        mesh=mesh,
        out_type=jax.ShapeDtypeStruct((TOTAL * 2,), jnp.float32),
        scratch_types=[
            pltpu.VMEM((4 * C,), jnp.float32),
            pltpu.VMEM((2 * C,), jnp.int32),
            pltpu.VMEM((2 * C,), jnp.float32),
            pltpu.SemaphoreType.DMA,
        ],
    )
    def gather_kernel(ev_hbm, flow_hbm, out_hbm, ev_v, idx_v, rows_v, sem):
        wid = lax.axis_index("s") * NC + lax.axis_index("c")
        tiles_per_batch = NW // B
        batch = wid // tiles_per_batch
        tile_base = wid * per_tile

        lane = lax.iota(jnp.int32, _LANES)
        ix_x = (((lane >> 1) * 4) + 1) & 15
        ix_y = ix_x + 1
        in_lo = lane < 8
        chan_off = ((batch * (2 * HW)).astype(jnp.float32)
                    + (lane & 1).astype(jnp.float32) * float(HW))

        for base in bases:
            evbase = tile_base + base
            pltpu.sync_copy(ev_hbm.at[pl.ds(evbase * 4, C * 4)], ev_v)

            def body(i, carry):
                v0 = ev_v[pl.ds(32 * i, _LANES)]
                v1 = ev_v[pl.ds(32 * i + _LANES, _LANES)]
                xv = jnp.where(in_lo, _take16(v0, ix_x), _take16(v1, ix_x))
                yv = jnp.where(in_lo, _take16(v0, ix_y), _take16(v1, ix_y))
                t = xv + yv * float(W) + chan_off
                idx_v[pl.ds(i * _LANES, _LANES)] = t.astype(jnp.int32)
                return carry

            lax.fori_loop(0, (2 * C) // _LANES, body, 0)

            copy = pltpu.async_copy(flow_hbm.at[idx_v], rows_v, sem)
            copy.wait()
            pltpu.sync_copy(rows_v, out_hbm.at[pl.ds(2 * evbase, 2 * C)])

    return gather_kernel


def kernel(flow_map, event_list, event_mask, dt_input, dt_gt):
    B, _, H, W = flow_map.shape
    N = event_list.shape[1]
    flow_flat = flow_map.reshape(B * 2 * H * W)
    ev_flat = event_list.reshape(B * N * 4)
    out = _build_gather_kernel(B, N, H, W)(ev_flat, flow_flat)
    return out.reshape(B, N, 2)


# speedup vs baseline: 2.5184x; 2.5184x over previous
import functools

import jax
import jax.numpy as jnp
from jax import lax
from jax.experimental import pallas as pl
from jax.experimental.pallas import tpu as pltpu
from jax.experimental.pallas import tpu_sc as plsc

_LANES = 16

_GATHER_DNUMS = lax.GatherDimensionNumbers(
    offset_dims=(), collapsed_slice_dims=(0,), start_index_map=(0,))


def _take16(vec, idx):
    return lax.gather(vec, idx[:, None], _GATHER_DNUMS, (1,),
                      mode=lax.GatherScatterMode.PROMISE_IN_BOUNDS)


@functools.lru_cache(maxsize=None)
def _build_gather_kernel(B, N, H, W):
    info = plsc.get_sparse_core_info()
    NC, NS = info.num_cores, info.num_subcores
    NW = NC * NS
    HW = H * W
    TOTAL = B * N
    assert TOTAL % NW == 0
    per_tile = TOTAL // NW

    C = min(12800, per_tile)
    assert C % _LANES == 0
    bases = [j * C for j in range(per_tile // C)]
    if per_tile % C:
        bases.append(per_tile - C)
    assert all(b % 8 == 0 for b in bases)

    mesh = plsc.VectorSubcoreMesh(core_axis_name="c", subcore_axis_name="s")

    @functools.partial(
        pl.kernel,
        mesh=mesh,
        out_type=jax.ShapeDtypeStruct((TOTAL * 2,), jnp.float32),
        scratch_types=[
            pltpu.VMEM((C,), jnp.float32),
            pltpu.VMEM((C,), jnp.float32),
            pltpu.VMEM((2 * C,), jnp.int32),
            pltpu.VMEM((2 * C,), jnp.float32),
            pltpu.SemaphoreType.DMA,
        ],
    )
    def gather_kernel(x_hbm, y_hbm, flow_hbm, out_hbm, x_v, y_v, idx_v, rows_v,
                      sem):
        wid = lax.axis_index("s") * NC + lax.axis_index("c")
        tiles_per_batch = NW // B
        batch = wid // tiles_per_batch
        tile_base = wid * per_tile

        lane = lax.iota(jnp.int32, _LANES)
        dup_lo = lane >> 1
        dup_hi = dup_lo + 8
        chan_off = ((batch * (2 * HW)).astype(jnp.float32)
                    + (lane & 1).astype(jnp.float32) * float(HW))

        for base in bases:
            evbase = tile_base + base
            pltpu.sync_copy(x_hbm.at[pl.ds(evbase, C)], x_v)
            pltpu.sync_copy(y_hbm.at[pl.ds(evbase, C)], y_v)

            def body(i, carry):
                xv = x_v[pl.ds(i * _LANES, _LANES)]
                yv = y_v[pl.ds(i * _LANES, _LANES)]
                t = xv + yv * float(W)
                lo = _take16(t, dup_lo) + chan_off
                hi = _take16(t, dup_hi) + chan_off
                idx_v[pl.ds(2 * i * _LANES, _LANES)] = lo.astype(jnp.int32)
                idx_v[pl.ds((2 * i + 1) * _LANES, _LANES)] = hi.astype(jnp.int32)
                return carry

            lax.fori_loop(0, C // _LANES, body, 0)

            copy = pltpu.async_copy(flow_hbm.at[idx_v], rows_v, sem)
            copy.wait()
            pltpu.sync_copy(rows_v, out_hbm.at[pl.ds(2 * evbase, 2 * C)])

    return gather_kernel


def kernel(flow_map, event_list, event_mask, dt_input, dt_gt):
    B, _, H, W = flow_map.shape
    N = event_list.shape[1]
    big = jnp.float32(3.4e38)
    flow_flat = jnp.minimum(flow_map, big).reshape(B * 2 * H * W)
    x_col = jnp.minimum(event_list[:, :, 1], big).reshape(B * N)
    y_col = jnp.minimum(event_list[:, :, 2], big).reshape(B * N)
    out = _build_gather_kernel(B, N, H, W)(x_col, y_col, flow_flat)
    return out.reshape(B, N, 2)


# speedup vs baseline: 11.1522x; 4.4282x over previous
import functools

import jax
import jax.numpy as jnp
from jax import lax
from jax.experimental import pallas as pl
from jax.experimental.pallas import tpu as pltpu
from jax.experimental.pallas import tpu_sc as plsc

_LANES = 16
_BLK = 128


@functools.lru_cache(maxsize=None)
def _build_gather_kernel(B, N, H, W):
    info = plsc.get_sparse_core_info()
    NC, NS = info.num_cores, info.num_subcores
    NW = NC * NS
    HW = H * W
    NP = -(-N // _BLK)
    TPB = NW // B
    BPT = -(-NP // TPB)
    CB = 100
    C = CB * _BLK
    n_chunks = -(-BPT // CB)

    mesh = plsc.VectorSubcoreMesh(core_axis_name="c", subcore_axis_name="s")

    @functools.partial(
        pl.kernel,
        mesh=mesh,
        out_type=jax.ShapeDtypeStruct((B * NP * 2 * _BLK,), jnp.float32),
        compiler_params=pltpu.CompilerParams(use_tc_tiling_on_sc=False),
        scratch_types=[
            pltpu.VMEM((C,), jnp.float32),
            pltpu.VMEM((C,), jnp.float32),
            pltpu.VMEM((2 * C,), jnp.int32),
            pltpu.VMEM((2 * C,), jnp.float32),
            pltpu.SemaphoreType.DMA,
        ],
    )
    def gather_kernel(x_hbm, y_hbm, flow_hbm, out_hbm, x_v, y_v, idx_v, rows_v,
                      sem):
        wid = lax.axis_index("s") * NC + lax.axis_index("c")
        batch = wid // TPB
        q = wid % TPB
        blk_lo = q * BPT
        blk_hi = jnp.minimum(blk_lo + BPT, NP)
        plane0 = batch * (2 * HW)

        for j in range(n_chunks):
            kb = jnp.minimum(blk_lo + j * CB, blk_hi - CB)
            ev_g = batch * N + kb * _BLK
            out_off = (batch * NP + kb) * (2 * _BLK)
            pltpu.sync_copy(x_hbm.at[pl.ds(ev_g, C)], x_v)
            pltpu.sync_copy(y_hbm.at[pl.ds(ev_g, C)], y_v)

            def body(i, carry):
                xv = x_v[pl.ds(i * _LANES, _LANES)]
                yv = y_v[pl.ds(i * _LANES, _LANES)]
                ti = (xv + yv * float(W)).astype(jnp.int32)
                ti = jnp.clip(ti, 0, HW - 1) + plane0
                pos = ((i >> 3) * (2 * _BLK)) + ((i & 7) * _LANES)
                idx_v[pl.ds(pos, _LANES)] = ti
                idx_v[pl.ds(pos + _BLK, _LANES)] = ti + HW
                return carry

            lax.fori_loop(0, C // _LANES, body, 0)

            pltpu.async_copy(flow_hbm.at[idx_v], rows_v, sem).wait()
            pltpu.sync_copy(rows_v, out_hbm.at[pl.ds(out_off, 2 * C)])

    return gather_kernel


def kernel(flow_map, event_list, event_mask, dt_input, dt_gt):
    B, _, H, W = flow_map.shape
    N = event_list.shape[1]
    NP = -(-N // _BLK)
    big = jnp.float32(3.4e38)
    flow_flat = jnp.minimum(flow_map.reshape(B * 2 * H * W), big)
    x_col = jnp.minimum(event_list[:, :, 1].reshape(B * N), big)
    y_col = jnp.minimum(event_list[:, :, 2].reshape(B * N), big)
    out = _build_gather_kernel(B, N, H, W)(x_col, y_col, flow_flat)
    res = out.reshape(B, NP, 2, _BLK).transpose(0, 1, 3, 2)
    return res.reshape(B, NP * _BLK, 2)[:, :N, :]


# speedup vs baseline: 12.4729x; 1.1184x over previous
import functools

import jax
import jax.numpy as jnp
from jax import lax
from jax.experimental import pallas as pl
from jax.experimental.pallas import tpu as pltpu
from jax.experimental.pallas import tpu_sc as plsc

_LANES = 16
_BLK = 128


@functools.lru_cache(maxsize=None)
def _build_gather_kernel(B, N, H, W):
    info = plsc.get_sparse_core_info()
    NC, NS = info.num_cores, info.num_subcores
    NW = NC * NS
    HW = H * W
    NP = -(-N // _BLK)
    TPB = NW // B
    BPT = -(-NP // TPB)
    CB = 96
    C = CB * _BLK
    n_chunks = -(-BPT // CB)

    mesh = plsc.VectorSubcoreMesh(core_axis_name="c", subcore_axis_name="s")

    @functools.partial(
        pl.kernel,
        mesh=mesh,
        out_type=jax.ShapeDtypeStruct((B * NP * 2 * _BLK,), jnp.float32),
        compiler_params=pltpu.CompilerParams(use_tc_tiling_on_sc=False),
        scratch_types=[
            pltpu.VMEM((C,), jnp.float32),
            pltpu.VMEM((C,), jnp.float32),
            pltpu.VMEM((2 * C,), jnp.int32),
            pltpu.VMEM((2 * C,), jnp.int32),
            pltpu.VMEM((2 * C,), jnp.float32),
            pltpu.VMEM((2 * C,), jnp.float32),
            pltpu.SemaphoreType.DMA,
            pltpu.SemaphoreType.DMA,
        ],
    )
    def gather_kernel(x_hbm, y_hbm, flow_hbm, out_hbm, x_v, y_v, idx0_v, idx1_v,
                      rows0_v, rows1_v, sem0, sem1):
        wid = lax.axis_index("s") * NC + lax.axis_index("c")
        batch = wid // TPB
        q = wid % TPB
        blk_lo = q * BPT
        blk_hi = jnp.minimum(blk_lo + BPT, NP)
        plane0 = batch * (2 * HW)
        idx_bufs = (idx0_v, idx1_v)
        rows_bufs = (rows0_v, rows1_v)
        sems = (sem0, sem1)

        def chunk_coords(j):
            kb = jnp.minimum(blk_lo + j * CB, blk_hi - CB)
            return kb, batch * N + kb * _BLK, (batch * NP + kb) * (2 * _BLK)

        def compute_idx(ev_g, idx_v):
            pltpu.sync_copy(x_hbm.at[pl.ds(ev_g, C)], x_v)
            pltpu.sync_copy(y_hbm.at[pl.ds(ev_g, C)], y_v)

            def body(i, carry):
                xv = x_v[pl.ds(i * _LANES, _LANES)]
                yv = y_v[pl.ds(i * _LANES, _LANES)]
                ti = (xv + yv * float(W)).astype(jnp.int32)
                ti = jnp.clip(ti, 0, HW - 1) + plane0
                pos = ((i >> 3) * (2 * _BLK)) + ((i & 7) * _LANES)
                idx_v[pl.ds(pos, _LANES)] = ti
                idx_v[pl.ds(pos + _BLK, _LANES)] = ti + HW
                return carry

            lax.fori_loop(0, C // _LANES, body, 0)

        _, ev_g, out_prev = chunk_coords(0)
        compute_idx(ev_g, idx_bufs[0])
        gather = pltpu.async_copy(flow_hbm.at[idx_bufs[0]], rows_bufs[0], sems[0])
        for j in range(1, n_chunks):
            b = j & 1
            _, ev_g, out_off = chunk_coords(j)
            compute_idx(ev_g, idx_bufs[b])
            gather.wait()
            gather = pltpu.async_copy(flow_hbm.at[idx_bufs[b]], rows_bufs[b],
                                      sems[b])
            pltpu.sync_copy(rows_bufs[1 - b], out_hbm.at[pl.ds(out_prev, 2 * C)])
            out_prev = out_off
        gather.wait()
        pltpu.sync_copy(rows_bufs[(n_chunks - 1) & 1],
                        out_hbm.at[pl.ds(out_prev, 2 * C)])

    return gather_kernel


def kernel(flow_map, event_list, event_mask, dt_input, dt_gt):
    B, _, H, W = flow_map.shape
    N = event_list.shape[1]
    NP = -(-N // _BLK)
    big = jnp.float32(3.4e38)
    flow_flat = jnp.minimum(flow_map.reshape(B * 2 * H * W), big)
    x_col = jnp.minimum(event_list[:, :, 1].reshape(B * N), big)
    y_col = jnp.minimum(event_list[:, :, 2].reshape(B * N), big)
    out = _build_gather_kernel(B, N, H, W)(x_col, y_col, flow_flat)
    res = out.reshape(B, NP, 2, _BLK).transpose(0, 1, 3, 2)
    return res.reshape(B, NP * _BLK, 2)[:, :N, :]


# speedup vs baseline: 12.6592x; 1.0149x over previous
import functools

import jax
import jax.numpy as jnp
from jax import lax
from jax.experimental import pallas as pl
from jax.experimental.pallas import tpu as pltpu
from jax.experimental.pallas import tpu_sc as plsc

_LANES = 16
_BLK = 128


@functools.lru_cache(maxsize=None)
def _build_gather_kernel(B, N, H, W):
    info = plsc.get_sparse_core_info()
    NC, NS = info.num_cores, info.num_subcores
    NW = NC * NS
    HW = H * W
    NP = -(-N // _BLK)
    TPB = NW // B
    BPT = -(-NP // TPB)
    CB = 100
    C = CB * _BLK
    n_chunks = -(-BPT // CB)

    mesh = plsc.VectorSubcoreMesh(core_axis_name="c", subcore_axis_name="s")

    @functools.partial(
        pl.kernel,
        mesh=mesh,
        out_type=jax.ShapeDtypeStruct((B * NP * 2 * _BLK,), jnp.float32),
        compiler_params=pltpu.CompilerParams(use_tc_tiling_on_sc=False),
        scratch_types=[
            pltpu.VMEM((C,), jnp.float32),
            pltpu.VMEM((C,), jnp.float32),
            pltpu.VMEM((2 * C,), jnp.int32),
            pltpu.VMEM((2 * C,), jnp.int32),
            pltpu.VMEM((2 * C,), jnp.float32),
            pltpu.VMEM((2 * C,), jnp.float32),
            pltpu.SemaphoreType.DMA,
            pltpu.SemaphoreType.DMA,
        ],
    )
    def gather_kernel(x_hbm, y_hbm, flow_hbm, out_hbm, x_v, y_v, idx0_v, idx1_v,
                      rows0_v, rows1_v, sem0, sem1):
        wid = lax.axis_index("s") * NC + lax.axis_index("c")
        batch = wid // TPB
        q = wid % TPB
        blk_lo = q * BPT
        blk_hi = jnp.minimum(blk_lo + BPT, NP)
        plane0 = batch * (2 * HW)
        idx_bufs = (idx0_v, idx1_v)
        rows_bufs = (rows0_v, rows1_v)
        sems = (sem0, sem1)

        def chunk_coords(j):
            kb = jnp.minimum(blk_lo + j * CB, blk_hi - CB)
            return kb, batch * N + kb * _BLK, (batch * NP + kb) * (2 * _BLK)

        def compute_idx(ev_g, idx_v):
            pltpu.sync_copy(x_hbm.at[pl.ds(ev_g, C)], x_v)
            pltpu.sync_copy(y_hbm.at[pl.ds(ev_g, C)], y_v)

            def body(i, carry):
                xv = x_v[pl.ds(i * _LANES, _LANES)]
                yv = y_v[pl.ds(i * _LANES, _LANES)]
                ti = (xv + yv * float(W)).astype(jnp.int32)
                ti = jnp.clip(ti, 0, HW - 1) + plane0
                pos = ((i >> 3) * (2 * _BLK)) + ((i & 7) * _LANES)
                idx_v[pl.ds(pos, _LANES)] = ti
                idx_v[pl.ds(pos + _BLK, _LANES)] = ti + HW
                return carry

            lax.fori_loop(0, C // _LANES, body, 0)

        _, ev_g, out_prev = chunk_coords(0)
        compute_idx(ev_g, idx_bufs[0])
        gather = pltpu.async_copy(flow_hbm.at[idx_bufs[0]], rows_bufs[0], sems[0])
        for j in range(1, n_chunks):
            b = j & 1
            _, ev_g, out_off = chunk_coords(j)
            compute_idx(ev_g, idx_bufs[b])
            gather_new = pltpu.async_copy(flow_hbm.at[idx_bufs[b]], rows_bufs[b],
                                          sems[b])
            gather.wait()
            gather = gather_new
            pltpu.sync_copy(rows_bufs[1 - b], out_hbm.at[pl.ds(out_prev, 2 * C)])
            out_prev = out_off
        gather.wait()
        pltpu.sync_copy(rows_bufs[(n_chunks - 1) & 1],
                        out_hbm.at[pl.ds(out_prev, 2 * C)])

    return gather_kernel


def kernel(flow_map, event_list, event_mask, dt_input, dt_gt):
    B, _, H, W = flow_map.shape
    N = event_list.shape[1]
    NP = -(-N // _BLK)
    big = jnp.float32(3.4e38)
    flow_flat = jnp.minimum(flow_map.reshape(B * 2 * H * W), big)
    x_col = jnp.minimum(event_list[:, :, 1].reshape(B * N), big)
    y_col = jnp.minimum(event_list[:, :, 2].reshape(B * N), big)
    out = _build_gather_kernel(B, N, H, W)(x_col, y_col, flow_flat)
    res = out.reshape(B, NP, 2, _BLK).transpose(0, 1, 3, 2)
    return res.reshape(B, NP * _BLK, 2)[:, :N, :]


# speedup vs baseline: 12.8556x; 1.0155x over previous
import functools

import jax
import jax.numpy as jnp
from jax import lax
from jax.experimental import pallas as pl
from jax.experimental.pallas import tpu as pltpu
from jax.experimental.pallas import tpu_sc as plsc

_LANES = 16
_BLK = 128


@functools.lru_cache(maxsize=None)
def _build_gather_kernel(B, N, H, W):
    info = plsc.get_sparse_core_info()
    NC, NS = info.num_cores, info.num_subcores
    NW = NC * NS
    HW = H * W
    NP = -(-N // _BLK)
    TPB = NW // B
    BPT = -(-NP // TPB)
    CB = 100
    C = CB * _BLK
    n_chunks = -(-BPT // CB)

    mesh = plsc.VectorSubcoreMesh(core_axis_name="c", subcore_axis_name="s")

    @functools.partial(
        pl.kernel,
        mesh=mesh,
        out_type=jax.ShapeDtypeStruct((B * NP * 2 * _BLK,), jnp.float32),
        compiler_params=pltpu.CompilerParams(use_tc_tiling_on_sc=False),
        scratch_types=[
            pltpu.VMEM((C,), jnp.float32),
            pltpu.VMEM((C,), jnp.float32),
            pltpu.VMEM((2 * C,), jnp.int32),
            pltpu.VMEM((2 * C,), jnp.int32),
            pltpu.VMEM((2 * C,), jnp.float32),
            pltpu.VMEM((2 * C,), jnp.float32),
            pltpu.SemaphoreType.DMA,
            pltpu.SemaphoreType.DMA,
        ],
    )
    def gather_kernel(x_hbm, y_hbm, flow_hbm, out_hbm, x_v, y_v, idx0_v, idx1_v,
                      rows0_v, rows1_v, sem0, sem1):
        wid = lax.axis_index("s") * NC + lax.axis_index("c")
        batch = wid // TPB
        q = wid % TPB
        blk_lo = q * BPT
        blk_hi = jnp.minimum(blk_lo + BPT, NP)
        plane0 = batch * (2 * HW)
        idx_bufs = (idx0_v, idx1_v)
        rows_bufs = (rows0_v, rows1_v)
        sems = (sem0, sem1)

        def chunk_coords(j):
            kb = jnp.minimum(blk_lo + j * CB, blk_hi - CB)
            return kb, batch * N + kb * _BLK, (batch * NP + kb) * (2 * _BLK)

        def compute_idx(ev_g, idx_v):
            pltpu.sync_copy(x_hbm.at[pl.ds(ev_g, C)], x_v)
            pltpu.sync_copy(y_hbm.at[pl.ds(ev_g, C)], y_v)

            def body(i, carry):
                xv = x_v[pl.ds(i * _LANES, _LANES)]
                yv = y_v[pl.ds(i * _LANES, _LANES)]
                ti = (xv + yv * float(W)).astype(jnp.int32)
                ti = jnp.clip(ti, 0, HW - 1) + plane0
                pos = ((i >> 3) * (2 * _BLK)) + ((i & 7) * _LANES)
                idx_v[pl.ds(pos, _LANES)] = ti
                idx_v[pl.ds(pos + _BLK, _LANES)] = ti + HW
                return carry

            lax.fori_loop(0, C // _LANES, body, 0)

        _, ev_g, out_prev = chunk_coords(0)
        compute_idx(ev_g, idx_bufs[0])
        gather = pltpu.async_copy(flow_hbm.at[idx_bufs[0]], rows_bufs[0], sems[0])
        for j in range(1, n_chunks):
            b = j & 1
            _, ev_g, out_off = chunk_coords(j)
            compute_idx(ev_g, idx_bufs[b])
            gather_new = pltpu.async_copy(flow_hbm.at[idx_bufs[b]], rows_bufs[b],
                                          sems[b])
            gather.wait()
            gather = gather_new
            pltpu.sync_copy(rows_bufs[1 - b], out_hbm.at[pl.ds(out_prev, 2 * C)])
            out_prev = out_off
        gather.wait()
        pltpu.sync_copy(rows_bufs[(n_chunks - 1) & 1],
                        out_hbm.at[pl.ds(out_prev, 2 * C)])

    return gather_kernel


def kernel(flow_map, event_list, event_mask, dt_input, dt_gt):
    B, _, H, W = flow_map.shape
    N = event_list.shape[1]
    NP = -(-N // _BLK)
    big = jnp.float32(3.4e38)
    flow_flat = flow_map.reshape(B * 2 * H * W)
    x_col = jnp.minimum(event_list[:, :, 1].reshape(B * N), big)
    y_col = jnp.minimum(event_list[:, :, 2].reshape(B * N), big)
    out = _build_gather_kernel(B, N, H, W)(x_col, y_col, flow_flat)
    res = out.reshape(B, NP, 2, _BLK).transpose(0, 1, 3, 2)
    return res.reshape(B, NP * _BLK, 2)[:, :N, :]


# speedup vs baseline: 12.9971x; 1.0110x over previous
import functools

import jax
import jax.numpy as jnp
from jax import lax
from jax.experimental import pallas as pl
from jax.experimental.pallas import tpu as pltpu
from jax.experimental.pallas import tpu_sc as plsc

_LANES = 16
_BLK = 128


@functools.lru_cache(maxsize=None)
def _build_gather_kernel(B, N, H, W):
    info = plsc.get_sparse_core_info()
    NC, NS = info.num_cores, info.num_subcores
    NW = NC * NS
    HW = H * W
    NP = -(-N // _BLK)
    TPB = NW // B
    BPT = -(-NP // TPB)
    CB = 64
    NBUF = 3
    C = CB * _BLK
    n_chunks = -(-BPT // CB)

    mesh = plsc.VectorSubcoreMesh(core_axis_name="c", subcore_axis_name="s")

    @functools.partial(
        pl.kernel,
        mesh=mesh,
        out_type=jax.ShapeDtypeStruct((B * NP * 2 * _BLK,), jnp.float32),
        compiler_params=pltpu.CompilerParams(use_tc_tiling_on_sc=False),
        scratch_types=[
            pltpu.VMEM((C,), jnp.float32),
            pltpu.VMEM((C,), jnp.float32),
            *[pltpu.VMEM((2 * C,), jnp.int32) for _ in range(NBUF)],
            *[pltpu.VMEM((2 * C,), jnp.float32) for _ in range(NBUF)],
            *[pltpu.SemaphoreType.DMA for _ in range(NBUF)],
        ],
    )
    def gather_kernel(x_hbm, y_hbm, flow_hbm, out_hbm, x_v, y_v, *bufs):
        idx_bufs = bufs[:NBUF]
        rows_bufs = bufs[NBUF:2 * NBUF]
        sems = bufs[2 * NBUF:]
        wid = lax.axis_index("s") * NC + lax.axis_index("c")
        batch = wid // TPB
        q = wid % TPB
        blk_lo = q * BPT
        blk_hi = jnp.minimum(blk_lo + BPT, NP)
        plane0 = batch * (2 * HW)

        def chunk_coords(j):
            kb = jnp.minimum(blk_lo + j * CB, blk_hi - CB)
            return kb, batch * N + kb * _BLK, (batch * NP + kb) * (2 * _BLK)

        def compute_idx(ev_g, idx_v):
            pltpu.sync_copy(x_hbm.at[pl.ds(ev_g, C)], x_v)
            pltpu.sync_copy(y_hbm.at[pl.ds(ev_g, C)], y_v)

            def body(i, carry):
                xv = x_v[pl.ds(i * _LANES, _LANES)]
                yv = y_v[pl.ds(i * _LANES, _LANES)]
                ti = (xv + yv * float(W)).astype(jnp.int32)
                ti = jnp.clip(ti, 0, HW - 1) + plane0
                pos = ((i >> 3) * (2 * _BLK)) + ((i & 7) * _LANES)
                idx_v[pl.ds(pos, _LANES)] = ti
                idx_v[pl.ds(pos + _BLK, _LANES)] = ti + HW
                return carry

            lax.fori_loop(0, C // _LANES, body, 0)

        gathers = [None] * n_chunks
        out_offs = [None] * n_chunks

        def drain(k):
            gathers[k].wait()
            pltpu.sync_copy(rows_bufs[k % NBUF],
                            out_hbm.at[pl.ds(out_offs[k], 2 * C)])

        for j in range(n_chunks):
            b = j % NBUF
            _, ev_g, out_offs[j] = chunk_coords(j)
            compute_idx(ev_g, idx_bufs[b])
            gathers[j] = pltpu.async_copy(flow_hbm.at[idx_bufs[b]], rows_bufs[b],
                                          sems[b])
            if j >= NBUF - 1:
                drain(j - (NBUF - 1))
        for k in range(max(0, n_chunks - (NBUF - 1)), n_chunks):
            drain(k)

    return gather_kernel


def kernel(flow_map, event_list, event_mask, dt_input, dt_gt):
    B, _, H, W = flow_map.shape
    N = event_list.shape[1]
    NP = -(-N // _BLK)
    big = jnp.float32(3.4e38)
    flow_flat = flow_map.reshape(B * 2 * H * W)
    x_col = jnp.minimum(event_list[:, :, 1].reshape(B * N), big)
    y_col = jnp.minimum(event_list[:, :, 2].reshape(B * N), big)
    out = _build_gather_kernel(B, N, H, W)(x_col, y_col, flow_flat)
    res = out.reshape(B, NP, 2, _BLK).transpose(0, 1, 3, 2)
    return res.reshape(B, NP * _BLK, 2)[:, :N, :]
